# Initial kernel scaffold; baseline (speedup 1.0000x reference)
import jax
import jax.numpy as jnp
from jax import lax
from jax.experimental import pallas as pl
from jax.experimental.pallas import tpu as pltpu
from jax.experimental.pallas import tpu_sc as plsc

VOCAB = 1000000
EMB = 64
B = 16384
CTX = 20
NEG = 20

NC = 2
NS = 16
NW = NC * NS
PER_W = B // NW
E = 16
NCHUNK = PER_W // E
ROWS = E * CTX
SUB = 4
RS = ROWS // SUB


def _sc_body(ctx_hbm, tgt_hbm, neg_hbm, win_hbm, wout_hbm,
             pos_hbm, negsc_hbm,
             ctx_idx, neg_idx, tgt_idx,
             ctx_rows0, ctx_rows1, neg_rows0, neg_rows1,
             tgt_rows0, tgt_rows1,
             pos_buf, negsc_buf, sem0, sem1):
  c = lax.axis_index("c")
  s = lax.axis_index("s")
  w = c * NS + s

  pltpu.sync_copy(ctx_hbm.at[pl.ds(pl.multiple_of(w * (PER_W * CTX), 8),
                                   PER_W * CTX)], ctx_idx)
  pltpu.sync_copy(neg_hbm.at[pl.ds(pl.multiple_of(w * (PER_W * NEG), 8),
                                   PER_W * NEG)], neg_idx)
  pltpu.sync_copy(tgt_hbm.at[pl.ds(pl.multiple_of(w * PER_W, 8), PER_W)],
                  tgt_idx)

  def transfers(g, rows_ctx, rows_neg, rows_tgt, sem):
    ds = []
    for k in range(SUB):
      off = pl.multiple_of(g * ROWS + k * RS, 8)
      ds.append(pltpu.make_async_copy(
          win_hbm.at[ctx_idx.at[pl.ds(off, RS)]],
          rows_ctx.at[pl.ds(k * RS, RS)], sem))
      ds.append(pltpu.make_async_copy(
          wout_hbm.at[neg_idx.at[pl.ds(off, RS)]],
          rows_neg.at[pl.ds(k * RS, RS)], sem))
    ds.append(pltpu.make_async_copy(
        wout_hbm.at[tgt_idx.at[pl.ds(pl.multiple_of(g * E, 8), E)]],
        rows_tgt, sem))
    return ds

  def issue(g, rows_ctx, rows_neg, rows_tgt, sem):
    for d in transfers(g, rows_ctx, rows_neg, rows_tgt, sem):
      d.start()

  def drain(g, rows_ctx, rows_neg, rows_tgt, sem):
    for d in transfers(g, rows_ctx, rows_neg, rows_tgt, sem):
      d.wait()

  lane = lax.iota(jnp.int32, 16)
  row_base = [lane * CTX + cc for cc in range(CTX)]

  def compute(g, rows_ctx, rows_neg, rows_tgt):
    def dbody(d, carry):
      pos = carry[0]
      negacc = carry[1:]
      col = jnp.full((16,), d, dtype=jnp.int32)
      acc = plsc.load_gather(rows_ctx, [row_base[0], col])
      for cc in range(1, CTX):
        acc = acc + plsc.load_gather(rows_ctx, [row_base[cc], col])
      emb = acc * jnp.float32(1.0 / CTX)
      tgt = plsc.load_gather(rows_tgt, [lane, col])
      pos = pos + emb * tgt
      negacc = tuple(
          negacc[j] + emb * plsc.load_gather(rows_neg, [row_base[j], col])
          for j in range(NEG))
      return (pos,) + negacc

    zero = jnp.zeros((16,), jnp.float32)
    out = lax.fori_loop(0, EMB, dbody, (zero,) * (1 + NEG))
    eoff = pl.multiple_of(g * E, 8)
    pos_buf[pl.ds(eoff, E)] = out[0]
    for j in range(NEG):
      negsc_buf[j, pl.ds(eoff, E)] = out[1 + j]

  issue(0, ctx_rows0, neg_rows0, tgt_rows0, sem0)
  issue(1, ctx_rows1, neg_rows1, tgt_rows1, sem1)

  def tbody(t, carry):
    g0 = t * 2
    drain(g0, ctx_rows0, neg_rows0, tgt_rows0, sem0)
    compute(g0, ctx_rows0, neg_rows0, tgt_rows0)

    @pl.when(g0 + 2 < NCHUNK)
    def _():
      issue(g0 + 2, ctx_rows0, neg_rows0, tgt_rows0, sem0)

    g1 = g0 + 1
    drain(g1, ctx_rows1, neg_rows1, tgt_rows1, sem1)
    compute(g1, ctx_rows1, neg_rows1, tgt_rows1)

    @pl.when(g1 + 2 < NCHUNK)
    def _():
      issue(g1 + 2, ctx_rows1, neg_rows1, tgt_rows1, sem1)

    return carry

  lax.fori_loop(0, NCHUNK // 2, tbody, 0)

  pltpu.sync_copy(pos_buf,
                  pos_hbm.at[pl.ds(pl.multiple_of(w * PER_W, 8), PER_W)])
  pltpu.sync_copy(negsc_buf, negsc_hbm.at[w])


_sc_call = pl.kernel(
    _sc_body,
    out_type=(jax.ShapeDtypeStruct((B,), jnp.float32),
              jax.ShapeDtypeStruct((NW, NEG, PER_W), jnp.float32)),
    mesh=plsc.VectorSubcoreMesh(core_axis_name="c", subcore_axis_name="s"),
    scratch_types=[
        pltpu.VMEM((PER_W * CTX,), jnp.int32),
        pltpu.VMEM((PER_W * NEG,), jnp.int32),
        pltpu.VMEM((PER_W,), jnp.int32),
        pltpu.VMEM((ROWS, EMB), jnp.float32),
        pltpu.VMEM((ROWS, EMB), jnp.float32),
        pltpu.VMEM((ROWS, EMB), jnp.float32),
        pltpu.VMEM((ROWS, EMB), jnp.float32),
        pltpu.VMEM((E, EMB), jnp.float32),
        pltpu.VMEM((E, EMB), jnp.float32),
        pltpu.VMEM((PER_W,), jnp.float32),
        pltpu.VMEM((NEG, PER_W), jnp.float32),
        pltpu.SemaphoreType.DMA,
        pltpu.SemaphoreType.DMA,
    ],
)


def _tail_body(pos_ref, w_ref, neg_ref, out_ref):
  pos = pos_ref[...]
  wv = w_ref[...]
  neg = neg_ref[...]
  pos_l = jnp.log(jax.nn.sigmoid(pos) + 1e-10)
  neg_l = jnp.log(jax.nn.sigmoid(-neg) + 1e-10)
  total = jnp.sum(pos_l * wv) + jnp.sum(neg_l)
  out_ref[0, 0] = -total * jnp.float32(1.0 / B)


_tail_call = pl.pallas_call(
    _tail_body,
    out_shape=jax.ShapeDtypeStruct((1, 1), jnp.float32),
    out_specs=pl.BlockSpec(memory_space=pltpu.SMEM),
)


def kernel(contexts, target, negatives, weights, W_in, W_out):
  ctx_flat = contexts.astype(jnp.int32).reshape(-1)
  neg_flat = negatives.astype(jnp.int32).reshape(-1)
  tgt = target.astype(jnp.int32)
  pos_score, neg_score = _sc_call(ctx_flat, tgt, neg_flat, W_in, W_out)
  loss = _tail_call(pos_score.reshape(128, 128),
                    weights.reshape(128, 128),
                    neg_score.reshape(B * NEG // 128, 128))
  return loss.reshape(())


# speedup vs baseline: 3.5147x; 3.5147x over previous
import jax
import jax.numpy as jnp
from jax import lax
from jax.experimental import pallas as pl
from jax.experimental.pallas import tpu as pltpu
from jax.experimental.pallas import tpu_sc as plsc

VOCAB = 1000000
EMB = 64
B = 16384
CTX = 20
NEG = 20

NC = 2
NS = 16
NW = NC * NS
PER_W = B // NW
E = 16
NCHUNK = PER_W // E
ROWS = E * CTX
SUB = 4
RS = ROWS // SUB


def _sc_body(ctx_hbm, tgt_hbm, neg_hbm, win_hbm, wout_hbm,
             pos_hbm, negsc_hbm,
             ctx_idx, neg_idx, tgt_idx,
             ctx_rows0, ctx_rows1, neg_rows0, neg_rows1,
             tgt_rows0, tgt_rows1,
             pos_buf, negsc_buf, sem0, sem1):
  c = lax.axis_index("c")
  s = lax.axis_index("s")
  w = c * NS + s

  pltpu.sync_copy(ctx_hbm.at[pl.ds(pl.multiple_of(w * (PER_W * CTX), 8),
                                   PER_W * CTX)], ctx_idx)
  pltpu.sync_copy(neg_hbm.at[pl.ds(pl.multiple_of(w * (PER_W * NEG), 8),
                                   PER_W * NEG)], neg_idx)
  pltpu.sync_copy(tgt_hbm.at[pl.ds(pl.multiple_of(w * PER_W, 8), PER_W)],
                  tgt_idx)

  def transfers(g, rows_ctx, rows_neg, rows_tgt, sem):
    ds = []
    for k in range(SUB):
      off = pl.multiple_of(g * ROWS + k * RS, 8)
      ds.append(pltpu.make_async_copy(
          win_hbm.at[ctx_idx.at[pl.ds(off, RS)]],
          rows_ctx.at[pl.ds(k * RS, RS)], sem))
      ds.append(pltpu.make_async_copy(
          wout_hbm.at[neg_idx.at[pl.ds(off, RS)]],
          rows_neg.at[pl.ds(k * RS, RS)], sem))
    ds.append(pltpu.make_async_copy(
        wout_hbm.at[tgt_idx.at[pl.ds(pl.multiple_of(g * E, 8), E)]],
        rows_tgt, sem))
    return ds

  def issue(g, rows_ctx, rows_neg, rows_tgt, sem):
    for d in transfers(g, rows_ctx, rows_neg, rows_tgt, sem):
      d.start()

  def drain(g, rows_ctx, rows_neg, rows_tgt, sem):
    for d in transfers(g, rows_ctx, rows_neg, rows_tgt, sem):
      d.wait()

  lane = lax.iota(jnp.int32, 16)
  row_base = [lane * CTX + cc for cc in range(CTX)]

  def compute(g, rows_ctx, rows_neg, rows_tgt):
    def dbody(d, carry):
      pos = carry[0]
      negacc = carry[1:]
      col = jnp.full((16,), d, dtype=jnp.int32)
      acc = plsc.load_gather(rows_ctx, [row_base[0], col])
      for cc in range(1, CTX):
        acc = acc + plsc.load_gather(rows_ctx, [row_base[cc], col])
      emb = acc * jnp.float32(1.0 / CTX)
      tgt = plsc.load_gather(rows_tgt, [lane, col])
      pos = pos + emb * tgt
      negacc = tuple(
          negacc[j] + emb * plsc.load_gather(rows_neg, [row_base[j], col])
          for j in range(NEG))
      return (pos,) + negacc

    zero = jnp.zeros((16,), jnp.float32)
    out = lax.fori_loop(0, EMB, dbody, (zero,) * (1 + NEG))
    eoff = pl.multiple_of(g * E, 8)
    pos_buf[pl.ds(eoff, E)] = out[0]
    for j in range(NEG):
      negsc_buf[j, pl.ds(eoff, E)] = out[1 + j]

  issue(0, ctx_rows0, neg_rows0, tgt_rows0, sem0)
  issue(1, ctx_rows1, neg_rows1, tgt_rows1, sem1)

  def tbody(t, carry):
    g0 = t * 2
    drain(g0, ctx_rows0, neg_rows0, tgt_rows0, sem0)
    compute(g0, ctx_rows0, neg_rows0, tgt_rows0)

    @pl.when(g0 + 2 < NCHUNK)
    def _():
      issue(g0 + 2, ctx_rows0, neg_rows0, tgt_rows0, sem0)

    g1 = g0 + 1
    drain(g1, ctx_rows1, neg_rows1, tgt_rows1, sem1)
    compute(g1, ctx_rows1, neg_rows1, tgt_rows1)

    @pl.when(g1 + 2 < NCHUNK)
    def _():
      issue(g1 + 2, ctx_rows1, neg_rows1, tgt_rows1, sem1)

    return carry

  lax.fori_loop(0, NCHUNK // 2, tbody, 0)

  pltpu.sync_copy(pos_buf,
                  pos_hbm.at[pl.ds(pl.multiple_of(w * PER_W, 8), PER_W)])
  pltpu.sync_copy(negsc_buf, negsc_hbm.at[w])


_sc_call = pl.kernel(
    _sc_body,
    out_type=(jax.ShapeDtypeStruct((B,), jnp.float32),
              jax.ShapeDtypeStruct((NW, NEG, PER_W), jnp.float32)),
    mesh=plsc.VectorSubcoreMesh(core_axis_name="c", subcore_axis_name="s"),
    compiler_params=pltpu.CompilerParams(needs_layout_passes=False,
                                         use_tc_tiling_on_sc=False),
    scratch_types=[
        pltpu.VMEM((PER_W * CTX,), jnp.int32),
        pltpu.VMEM((PER_W * NEG,), jnp.int32),
        pltpu.VMEM((PER_W,), jnp.int32),
        pltpu.VMEM((ROWS, EMB), jnp.float32),
        pltpu.VMEM((ROWS, EMB), jnp.float32),
        pltpu.VMEM((ROWS, EMB), jnp.float32),
        pltpu.VMEM((ROWS, EMB), jnp.float32),
        pltpu.VMEM((E, EMB), jnp.float32),
        pltpu.VMEM((E, EMB), jnp.float32),
        pltpu.VMEM((PER_W,), jnp.float32),
        pltpu.VMEM((NEG, PER_W), jnp.float32),
        pltpu.SemaphoreType.DMA,
        pltpu.SemaphoreType.DMA,
    ],
)


def _tail_body(pos_ref, w_ref, neg_ref, out_ref):
  pos = pos_ref[...]
  wv = w_ref[...]
  neg = neg_ref[...]
  pos_l = jnp.log(jax.nn.sigmoid(pos) + 1e-10)
  neg_l = jnp.log(jax.nn.sigmoid(-neg) + 1e-10)
  total = jnp.sum(pos_l * wv) + jnp.sum(neg_l)
  out_ref[0, 0] = -total * jnp.float32(1.0 / B)


_tail_call = pl.pallas_call(
    _tail_body,
    out_shape=jax.ShapeDtypeStruct((1, 1), jnp.float32),
    out_specs=pl.BlockSpec(memory_space=pltpu.SMEM),
)


def kernel(contexts, target, negatives, weights, W_in, W_out):
  ctx_flat = contexts.astype(jnp.int32).reshape(-1)
  neg_flat = negatives.astype(jnp.int32).reshape(-1)
  tgt = target.astype(jnp.int32)
  pos_score, neg_score = _sc_call(ctx_flat, tgt, neg_flat, W_in, W_out)
  loss = _tail_call(pos_score.reshape(128, 128),
                    weights.reshape(128, 128),
                    neg_score.reshape(B * NEG // 128, 128))
  return loss.reshape(())


# speedup vs baseline: 5.3105x; 1.5109x over previous
import jax
import jax.numpy as jnp
from jax import lax
from jax.experimental import pallas as pl
from jax.experimental.pallas import tpu as pltpu
from jax.experimental.pallas import tpu_sc as plsc

VOCAB = 1000000
EMB = 64
B = 16384
CTX = 20
NEG = 20

NC = 2
NS = 16
NW = NC * NS
PER_W = B // NW
E = 16
NCHUNK = PER_W // E
ROWS = E * CTX
SUB = 4
RS = ROWS // SUB
K = EMB // 16


def _sc_body(ctx_hbm, tgt_hbm, neg_hbm, win_hbm, wout_hbm,
             posp_hbm, negp_hbm,
             ctx_idx, neg_idx, tgt_idx,
             ctx_rows0, ctx_rows1, neg_rows0, neg_rows1,
             tgt_rows0, tgt_rows1,
             posp0, posp1, negp0, negp1,
             sem0, sem1, osem0, osem1):
  c = lax.axis_index("c")
  s = lax.axis_index("s")
  w = c * NS + s

  pltpu.sync_copy(ctx_hbm.at[pl.ds(pl.multiple_of(w * (PER_W * CTX), 8),
                                   PER_W * CTX)], ctx_idx)
  pltpu.sync_copy(neg_hbm.at[pl.ds(pl.multiple_of(w * (PER_W * NEG), 8),
                                   PER_W * NEG)], neg_idx)
  pltpu.sync_copy(tgt_hbm.at[pl.ds(pl.multiple_of(w * PER_W, 8), PER_W)],
                  tgt_idx)

  def in_transfers(g, rows_ctx, rows_neg, rows_tgt, sem):
    ds = []
    for k in range(SUB):
      off = pl.multiple_of(g * ROWS + k * RS, 8)
      ds.append(pltpu.make_async_copy(
          win_hbm.at[ctx_idx.at[pl.ds(off, RS)]],
          rows_ctx.at[pl.ds(k * RS, RS)], sem))
      ds.append(pltpu.make_async_copy(
          wout_hbm.at[neg_idx.at[pl.ds(off, RS)]],
          rows_neg.at[pl.ds(k * RS, RS)], sem))
    ds.append(pltpu.make_async_copy(
        wout_hbm.at[tgt_idx.at[pl.ds(pl.multiple_of(g * E, 8), E)]],
        rows_tgt, sem))
    return ds

  def out_transfers(g, posp, negp, osem):
    base = pl.multiple_of(w * PER_W + g * E, 8)
    return [
        pltpu.make_async_copy(posp, posp_hbm.at[pl.ds(base, E)], osem),
        pltpu.make_async_copy(negp, negp_hbm.at[pl.ds(base, E)], osem),
    ]

  def compute(g, rows_ctx, rows_neg, rows_tgt, posp, negp):
    def ebody(e, carry):
      r0 = e * CTX
      acc = [rows_ctx[r0, pl.ds(k * 16, 16)] for k in range(K)]
      for cc in range(1, CTX):
        for k in range(K):
          acc[k] = acc[k] + rows_ctx[r0 + cc, pl.ds(k * 16, 16)]
      emb = [a * jnp.float32(1.0 / CTX) for a in acc]
      p = emb[0] * rows_tgt[e, pl.ds(0, 16)]
      for k in range(1, K):
        p = p + emb[k] * rows_tgt[e, pl.ds(k * 16, 16)]
      posp[e, :] = p
      for j in range(NEG):
        q = emb[0] * rows_neg[r0 + j, pl.ds(0, 16)]
        for k in range(1, K):
          q = q + emb[k] * rows_neg[r0 + j, pl.ds(k * 16, 16)]
        negp[e, j, :] = q
      return carry

    lax.fori_loop(0, E, ebody, 0)

  for d in in_transfers(0, ctx_rows0, neg_rows0, tgt_rows0, sem0):
    d.start()
  for d in in_transfers(1, ctx_rows1, neg_rows1, tgt_rows1, sem1):
    d.start()

  def tbody(t, carry):
    for par, (rows_ctx, rows_neg, rows_tgt, posp, negp, sem, osem) in enumerate(
        ((ctx_rows0, neg_rows0, tgt_rows0, posp0, negp0, sem0, osem0),
         (ctx_rows1, neg_rows1, tgt_rows1, posp1, negp1, sem1, osem1))):
      g = t * 2 + par
      for d in in_transfers(g, rows_ctx, rows_neg, rows_tgt, sem):
        d.wait()

      @pl.when(g >= 2)
      def _():
        for d in out_transfers(g - 2, posp, negp, osem):
          d.wait()

      compute(g, rows_ctx, rows_neg, rows_tgt, posp, negp)
      for d in out_transfers(g, posp, negp, osem):
        d.start()

      @pl.when(g + 2 < NCHUNK)
      def _():
        for d in in_transfers(g + 2, rows_ctx, rows_neg, rows_tgt, sem):
          d.start()

    return carry

  lax.fori_loop(0, NCHUNK // 2, tbody, 0)

  for d in out_transfers(NCHUNK - 2, posp0, negp0, osem0):
    d.wait()
  for d in out_transfers(NCHUNK - 1, posp1, negp1, osem1):
    d.wait()


_sc_call = pl.kernel(
    _sc_body,
    out_type=(jax.ShapeDtypeStruct((B, 16), jnp.float32),
              jax.ShapeDtypeStruct((B, NEG, 16), jnp.float32)),
    mesh=plsc.VectorSubcoreMesh(core_axis_name="c", subcore_axis_name="s"),
    compiler_params=pltpu.CompilerParams(needs_layout_passes=False,
                                         use_tc_tiling_on_sc=False),
    scratch_types=[
        pltpu.VMEM((PER_W * CTX,), jnp.int32),
        pltpu.VMEM((PER_W * NEG,), jnp.int32),
        pltpu.VMEM((PER_W,), jnp.int32),
        pltpu.VMEM((ROWS, EMB), jnp.float32),
        pltpu.VMEM((ROWS, EMB), jnp.float32),
        pltpu.VMEM((ROWS, EMB), jnp.float32),
        pltpu.VMEM((ROWS, EMB), jnp.float32),
        pltpu.VMEM((E, EMB), jnp.float32),
        pltpu.VMEM((E, EMB), jnp.float32),
        pltpu.VMEM((E, 16), jnp.float32),
        pltpu.VMEM((E, 16), jnp.float32),
        pltpu.VMEM((E, NEG, 16), jnp.float32),
        pltpu.VMEM((E, NEG, 16), jnp.float32),
        pltpu.SemaphoreType.DMA,
        pltpu.SemaphoreType.DMA,
        pltpu.SemaphoreType.DMA,
        pltpu.SemaphoreType.DMA,
    ],
)

NEG_ROWS = B * NEG * 16 // 128
NEG_BLK = 4096
NSTEP = NEG_ROWS // NEG_BLK
POS_ROWS = B * 16 // 128


def _tail_body(posp_ref, w_ref, negp_ref, out_ref):
  i = pl.program_id(0)
  rg = lax.broadcasted_iota(jnp.int32, (128, 128), 0) // 16
  cj = lax.broadcasted_iota(jnp.int32, (128, 128), 1)
  gmat = jnp.where((rg == cj) & (cj < 8), jnp.float32(1.0), jnp.float32(0.0))

  @pl.when(i == 0)
  def _():
    pos_s = jnp.dot(posp_ref[...], gmat, preferred_element_type=jnp.float32)
    pos_l = jnp.log(jax.nn.sigmoid(pos_s) + 1e-10) * w_ref[...]
    out_ref[0, 0] = jnp.sum(pos_l)

  mask = (lax.broadcasted_iota(jnp.int32, (NEG_BLK, 128), 1) < 8)
  neg_s = jnp.dot(negp_ref[...], gmat, preferred_element_type=jnp.float32)
  neg_l = jnp.where(mask, jnp.log(jax.nn.sigmoid(-neg_s) + 1e-10),
                    jnp.float32(0.0))
  out_ref[0, 0] += jnp.sum(neg_l)

  @pl.when(i == NSTEP - 1)
  def _():
    out_ref[0, 0] = out_ref[0, 0] * jnp.float32(-1.0 / B)


_tail_call = pl.pallas_call(
    _tail_body,
    grid=(NSTEP,),
    in_specs=[
        pl.BlockSpec((POS_ROWS, 128), lambda i: (0, 0)),
        pl.BlockSpec((POS_ROWS, 128), lambda i: (0, 0)),
        pl.BlockSpec((NEG_BLK, 128), lambda i: (i, 0)),
    ],
    out_specs=pl.BlockSpec(memory_space=pltpu.SMEM),
    out_shape=jax.ShapeDtypeStruct((1, 1), jnp.float32),
)


def kernel(contexts, target, negatives, weights, W_in, W_out):
  ctx_flat = contexts.astype(jnp.int32).reshape(-1)
  neg_flat = negatives.astype(jnp.int32).reshape(-1)
  tgt = target.astype(jnp.int32)
  pos_part, neg_part = _sc_call(ctx_flat, tgt, neg_flat, W_in, W_out)
  wpad = jnp.pad(weights.reshape(POS_ROWS, 8), ((0, 0), (0, 120)))
  loss = _tail_call(pos_part.reshape(POS_ROWS, 128), wpad,
                    neg_part.reshape(NEG_ROWS, 128))
  return loss.reshape(())


# speedup vs baseline: 5.3281x; 1.0033x over previous
import jax
import jax.numpy as jnp
from jax import lax
from jax.experimental import pallas as pl
from jax.experimental.pallas import tpu as pltpu
from jax.experimental.pallas import tpu_sc as plsc

VOCAB = 1000000
EMB = 64
B = 16384
CTX = 20
NEG = 20

NC = 2
NS = 16
NW = NC * NS
PER_W = B // NW
E = 16
NCHUNK = PER_W // E
ROWS = E * CTX
SUB = 4
RS = ROWS // SUB
K = EMB // 16


def _sc_body(ctxt_hbm, tgt_hbm, negt_hbm, win_hbm, wout_hbm,
             posp_hbm, negp_hbm,
             ctx_idx, neg_idx, tgt_idx, stage_c0, stage_c1, stage_n0, stage_n1,
             ctx_rows0, ctx_rows1, neg_rows0, neg_rows1,
             tgt_rows0, tgt_rows1,
             posp0, posp1, negp0, negp1,
             sem0, sem1, osem0, osem1):
  c = lax.axis_index("c")
  s = lax.axis_index("s")
  w = c * NS + s
  wbase = pl.multiple_of(w * PER_W, 8)

  pltpu.sync_copy(ctxt_hbm.at[:, pl.ds(wbase, PER_W)], ctx_idx)
  pltpu.sync_copy(negt_hbm.at[:, pl.ds(wbase, PER_W)], neg_idx)
  pltpu.sync_copy(tgt_hbm.at[pl.ds(wbase, PER_W)], tgt_idx)

  def stage(g, stage_c, stage_n):
    for cc in range(CTX):
      stage_c[pl.ds(cc * E, E)] = ctx_idx[cc, pl.ds(g * E, E)]
      stage_n[pl.ds(cc * E, E)] = neg_idx[cc, pl.ds(g * E, E)]

  def in_transfers(g, stage_c, stage_n, rows_ctx, rows_neg, rows_tgt, sem):
    ds = []
    for k in range(SUB):
      ds.append(pltpu.make_async_copy(
          win_hbm.at[stage_c.at[pl.ds(k * RS, RS)]],
          rows_ctx.at[pl.ds(k * RS, RS)], sem))
      ds.append(pltpu.make_async_copy(
          wout_hbm.at[stage_n.at[pl.ds(k * RS, RS)]],
          rows_neg.at[pl.ds(k * RS, RS)], sem))
    ds.append(pltpu.make_async_copy(
        wout_hbm.at[tgt_idx.at[pl.ds(pl.multiple_of(g * E, 8), E)]],
        rows_tgt, sem))
    return ds

  def out_transfers(g, posp, negp, osem):
    base = pl.multiple_of(w * PER_W + g * E, 8)
    return [
        pltpu.make_async_copy(posp, posp_hbm.at[pl.ds(base, E)], osem),
        pltpu.make_async_copy(negp, negp_hbm.at[pl.ds(base, E)], osem),
    ]

  def compute(g, rows_ctx, rows_neg, rows_tgt, posp, negp):
    def ebody(e, carry):
      acc = [rows_ctx[e, pl.ds(k * 16, 16)] for k in range(K)]
      for cc in range(1, CTX):
        for k in range(K):
          acc[k] = acc[k] + rows_ctx[cc * E + e, pl.ds(k * 16, 16)]
      emb = [a * jnp.float32(1.0 / CTX) for a in acc]
      p = emb[0] * rows_tgt[e, pl.ds(0, 16)]
      for k in range(1, K):
        p = p + emb[k] * rows_tgt[e, pl.ds(k * 16, 16)]
      posp[e, :] = p
      for j in range(NEG):
        q = emb[0] * rows_neg[j * E + e, pl.ds(0, 16)]
        for k in range(1, K):
          q = q + emb[k] * rows_neg[j * E + e, pl.ds(k * 16, 16)]
        negp[e, j, :] = q
      return carry

    lax.fori_loop(0, E, ebody, 0)

  stage(0, stage_c0, stage_n0)
  for d in in_transfers(0, stage_c0, stage_n0,
                        ctx_rows0, neg_rows0, tgt_rows0, sem0):
    d.start()
  stage(1, stage_c1, stage_n1)
  for d in in_transfers(1, stage_c1, stage_n1,
                        ctx_rows1, neg_rows1, tgt_rows1, sem1):
    d.start()

  def tbody(t, carry):
    for par, (stage_c, stage_n, rows_ctx, rows_neg, rows_tgt, posp, negp,
              sem, osem) in enumerate(
        ((stage_c0, stage_n0, ctx_rows0, neg_rows0, tgt_rows0, posp0, negp0,
          sem0, osem0),
         (stage_c1, stage_n1, ctx_rows1, neg_rows1, tgt_rows1, posp1, negp1,
          sem1, osem1))):
      g = t * 2 + par
      for d in in_transfers(g, stage_c, stage_n,
                            rows_ctx, rows_neg, rows_tgt, sem):
        d.wait()

      @pl.when(g >= 2)
      def _():
        for d in out_transfers(g - 2, posp, negp, osem):
          d.wait()

      compute(g, rows_ctx, rows_neg, rows_tgt, posp, negp)
      for d in out_transfers(g, posp, negp, osem):
        d.start()

      @pl.when(g + 2 < NCHUNK)
      def _():
        stage(g + 2, stage_c, stage_n)
        for d in in_transfers(g + 2, stage_c, stage_n,
                              rows_ctx, rows_neg, rows_tgt, sem):
          d.start()

    return carry

  lax.fori_loop(0, NCHUNK // 2, tbody, 0)

  for d in out_transfers(NCHUNK - 2, posp0, negp0, osem0):
    d.wait()
  for d in out_transfers(NCHUNK - 1, posp1, negp1, osem1):
    d.wait()


_sc_call = pl.kernel(
    _sc_body,
    out_type=(jax.ShapeDtypeStruct((B, 16), jnp.float32),
              jax.ShapeDtypeStruct((B, NEG, 16), jnp.float32)),
    mesh=plsc.VectorSubcoreMesh(core_axis_name="c", subcore_axis_name="s"),
    compiler_params=pltpu.CompilerParams(needs_layout_passes=False,
                                         use_tc_tiling_on_sc=False),
    scratch_types=[
        pltpu.VMEM((CTX, PER_W), jnp.int32),
        pltpu.VMEM((NEG, PER_W), jnp.int32),
        pltpu.VMEM((PER_W,), jnp.int32),
        pltpu.VMEM((ROWS,), jnp.int32),
        pltpu.VMEM((ROWS,), jnp.int32),
        pltpu.VMEM((ROWS,), jnp.int32),
        pltpu.VMEM((ROWS,), jnp.int32),
        pltpu.VMEM((ROWS, EMB), jnp.float32),
        pltpu.VMEM((ROWS, EMB), jnp.float32),
        pltpu.VMEM((ROWS, EMB), jnp.float32),
        pltpu.VMEM((ROWS, EMB), jnp.float32),
        pltpu.VMEM((E, EMB), jnp.float32),
        pltpu.VMEM((E, EMB), jnp.float32),
        pltpu.VMEM((E, 16), jnp.float32),
        pltpu.VMEM((E, 16), jnp.float32),
        pltpu.VMEM((E, NEG, 16), jnp.float32),
        pltpu.VMEM((E, NEG, 16), jnp.float32),
        pltpu.SemaphoreType.DMA,
        pltpu.SemaphoreType.DMA,
        pltpu.SemaphoreType.DMA,
        pltpu.SemaphoreType.DMA,
    ],
)

NEG_ROWS = B * NEG * 16 // 128
NEG_BLK = 4096
NSTEP = NEG_ROWS // NEG_BLK
POS_ROWS = B * 16 // 128


def _tail_body(posp_ref, w_ref, negp_ref, out_ref):
  i = pl.program_id(0)
  rg = lax.broadcasted_iota(jnp.int32, (128, 128), 0) // 16
  cj = lax.broadcasted_iota(jnp.int32, (128, 128), 1)
  gmat = jnp.where((rg == cj) & (cj < 8), jnp.float32(1.0), jnp.float32(0.0))

  @pl.when(i == 0)
  def _():
    pos_s = jnp.dot(posp_ref[...], gmat, preferred_element_type=jnp.float32)
    pos_l = jnp.log(jax.nn.sigmoid(pos_s) + 1e-10) * w_ref[...]
    out_ref[0, 0] = jnp.sum(pos_l)

  mask = (lax.broadcasted_iota(jnp.int32, (NEG_BLK, 128), 1) < 8)
  neg_s = jnp.dot(negp_ref[...], gmat, preferred_element_type=jnp.float32)
  neg_l = jnp.where(mask, jnp.log(jax.nn.sigmoid(-neg_s) + 1e-10),
                    jnp.float32(0.0))
  out_ref[0, 0] += jnp.sum(neg_l)

  @pl.when(i == NSTEP - 1)
  def _():
    out_ref[0, 0] = out_ref[0, 0] * jnp.float32(-1.0 / B)


_tail_call = pl.pallas_call(
    _tail_body,
    grid=(NSTEP,),
    in_specs=[
        pl.BlockSpec((POS_ROWS, 128), lambda i: (0, 0)),
        pl.BlockSpec((POS_ROWS, 128), lambda i: (0, 0)),
        pl.BlockSpec((NEG_BLK, 128), lambda i: (i, 0)),
    ],
    out_specs=pl.BlockSpec(memory_space=pltpu.SMEM),
    out_shape=jax.ShapeDtypeStruct((1, 1), jnp.float32),
)


def kernel(contexts, target, negatives, weights, W_in, W_out):
  ctx_t = contexts.astype(jnp.int32).T
  neg_t = negatives.astype(jnp.int32).T
  tgt = target.astype(jnp.int32)
  pos_part, neg_part = _sc_call(ctx_t, tgt, neg_t, W_in, W_out)
  wpad = jnp.pad(weights.reshape(POS_ROWS, 8), ((0, 0), (0, 120)))
  loss = _tail_call(pos_part.reshape(POS_ROWS, 128), wpad,
                    neg_part.reshape(NEG_ROWS, 128))
  return loss.reshape(())


# speedup vs baseline: 6.2040x; 1.1644x over previous
import jax
import jax.numpy as jnp
from jax import lax
from jax.experimental import pallas as pl
from jax.experimental.pallas import tpu as pltpu
from jax.experimental.pallas import tpu_sc as plsc

VOCAB = 1000000
EMB = 64
B = 16384
CTX = 20
NEG = 20

NC = 2
NS = 16
NW = NC * NS
PER_W = B // NW
E = 16
NCHUNK = PER_W // E
ROWS = E * CTX
SUB = 4
RS = ROWS // SUB
K = EMB // 16

HB = 1024
NBLK = -(-VOCAB // (2 * HB))
OUTR = NBLK * HB
VFULL = (VOCAB // (2 * HB)) * 2 * HB


def _tr_body(lo_ref, hi_ref, tail_ref, out_ref):
  sel = pl.program_id(0) == NBLK - 1
  lo = jnp.where(sel, tail_ref[...], lo_ref[...])
  out_ref[:, 0:64] = lo.T
  out_ref[:, 64:128] = hi_ref[...].T


_tr_call = pl.pallas_call(
    _tr_body,
    grid=(NBLK,),
    in_specs=[pl.BlockSpec((EMB, HB), lambda i: (0, jnp.minimum(2 * i, 974))),
              pl.BlockSpec((EMB, HB),
                           lambda i: (0, jnp.minimum(2 * i + 1, 975))),
              pl.BlockSpec((EMB, HB), lambda i: (0, 0))],
    out_specs=pl.BlockSpec((HB, 128), lambda i: (i, 0)),
    out_shape=jax.ShapeDtypeStruct((OUTR, 128), jnp.float32),
)


def _xform(v):
  big = v >= jnp.int32(VFULL)
  small = (lax.shift_left(lax.shift_right_logical(v, 11), 11)
           + lax.shift_left(v & 1023, 1)
           + (lax.shift_right_logical(v, 10) & 1))
  return jnp.where(big, v + v - jnp.int32(VFULL), small)


def _sc_body(ctxt_hbm, tgt_hbm, negt_hbm, win_hbm, wout_hbm,
             posp_hbm, negp_hbm,
             ctx_idx, neg_idx, tgt_idx, stage_c0, stage_c1, stage_n0, stage_n1,
             ctx_rows0, ctx_rows1, neg_rows0, neg_rows1,
             tgt_rows0, tgt_rows1,
             posp0, posp1, negp0, negp1,
             sem0, sem1, osem0, osem1):
  c = lax.axis_index("c")
  s = lax.axis_index("s")
  w = c * NS + s
  wbase = pl.multiple_of(w * PER_W, 8)

  pltpu.sync_copy(ctxt_hbm.at[:, pl.ds(wbase, PER_W)], ctx_idx)
  pltpu.sync_copy(negt_hbm.at[:, pl.ds(wbase, PER_W)], neg_idx)
  pltpu.sync_copy(tgt_hbm.at[pl.ds(wbase, PER_W)], tgt_idx)
  for i in range(PER_W // 16):
    tgt_idx[pl.ds(i * 16, 16)] = _xform(tgt_idx[pl.ds(i * 16, 16)])

  def stage(g, stage_c, stage_n):
    for cc in range(CTX):
      stage_c[pl.ds(cc * E, E)] = _xform(ctx_idx[cc, pl.ds(g * E, E)])
      stage_n[pl.ds(cc * E, E)] = _xform(neg_idx[cc, pl.ds(g * E, E)])

  def in_transfers(g, stage_c, stage_n, rows_ctx, rows_neg, rows_tgt, sem):
    ds = []
    for k in range(SUB):
      ds.append(pltpu.make_async_copy(
          win_hbm.at[stage_c.at[pl.ds(k * RS, RS)]],
          rows_ctx.at[pl.ds(k * RS, RS)], sem))
      ds.append(pltpu.make_async_copy(
          wout_hbm.at[stage_n.at[pl.ds(k * RS, RS)]],
          rows_neg.at[pl.ds(k * RS, RS)], sem))
    ds.append(pltpu.make_async_copy(
        wout_hbm.at[tgt_idx.at[pl.ds(pl.multiple_of(g * E, 8), E)]],
        rows_tgt, sem))
    return ds

  def out_transfers(g, posp, negp, osem):
    base = pl.multiple_of(w * PER_W + g * E, 8)
    return [
        pltpu.make_async_copy(posp, posp_hbm.at[pl.ds(base, E)], osem),
        pltpu.make_async_copy(negp, negp_hbm.at[pl.ds(base, E)], osem),
    ]

  def compute(g, rows_ctx, rows_neg, rows_tgt, posp, negp):
    def ebody(e, carry):
      acc = [rows_ctx[e, pl.ds(k * 16, 16)] for k in range(K)]
      for cc in range(1, CTX):
        for k in range(K):
          acc[k] = acc[k] + rows_ctx[cc * E + e, pl.ds(k * 16, 16)]
      emb = [a * jnp.float32(1.0 / CTX) for a in acc]
      p = emb[0] * rows_tgt[e, pl.ds(0, 16)]
      for k in range(1, K):
        p = p + emb[k] * rows_tgt[e, pl.ds(k * 16, 16)]
      posp[e, :] = p
      for j in range(NEG):
        q = emb[0] * rows_neg[j * E + e, pl.ds(0, 16)]
        for k in range(1, K):
          q = q + emb[k] * rows_neg[j * E + e, pl.ds(k * 16, 16)]
        negp[e, j, :] = q
      return carry

    lax.fori_loop(0, E, ebody, 0)

  stage(0, stage_c0, stage_n0)
  for d in in_transfers(0, stage_c0, stage_n0,
                        ctx_rows0, neg_rows0, tgt_rows0, sem0):
    d.start()
  stage(1, stage_c1, stage_n1)
  for d in in_transfers(1, stage_c1, stage_n1,
                        ctx_rows1, neg_rows1, tgt_rows1, sem1):
    d.start()

  def tbody(t, carry):
    for par, (stage_c, stage_n, rows_ctx, rows_neg, rows_tgt, posp, negp,
              sem, osem) in enumerate(
        ((stage_c0, stage_n0, ctx_rows0, neg_rows0, tgt_rows0, posp0, negp0,
          sem0, osem0),
         (stage_c1, stage_n1, ctx_rows1, neg_rows1, tgt_rows1, posp1, negp1,
          sem1, osem1))):
      g = t * 2 + par
      for d in in_transfers(g, stage_c, stage_n,
                            rows_ctx, rows_neg, rows_tgt, sem):
        d.wait()

      @pl.when(g >= 2)
      def _():
        for d in out_transfers(g - 2, posp, negp, osem):
          d.wait()

      compute(g, rows_ctx, rows_neg, rows_tgt, posp, negp)
      for d in out_transfers(g, posp, negp, osem):
        d.start()

      @pl.when(g + 2 < NCHUNK)
      def _():
        stage(g + 2, stage_c, stage_n)
        for d in in_transfers(g + 2, stage_c, stage_n,
                              rows_ctx, rows_neg, rows_tgt, sem):
          d.start()

    return carry

  lax.fori_loop(0, NCHUNK // 2, tbody, 0)

  for d in out_transfers(NCHUNK - 2, posp0, negp0, osem0):
    d.wait()
  for d in out_transfers(NCHUNK - 1, posp1, negp1, osem1):
    d.wait()


_sc_call = pl.kernel(
    _sc_body,
    out_type=(jax.ShapeDtypeStruct((B, 16), jnp.float32),
              jax.ShapeDtypeStruct((B, NEG, 16), jnp.float32)),
    mesh=plsc.VectorSubcoreMesh(core_axis_name="c", subcore_axis_name="s"),
    compiler_params=pltpu.CompilerParams(needs_layout_passes=False,
                                         use_tc_tiling_on_sc=False),
    scratch_types=[
        pltpu.VMEM((CTX, PER_W), jnp.int32),
        pltpu.VMEM((NEG, PER_W), jnp.int32),
        pltpu.VMEM((PER_W,), jnp.int32),
        pltpu.VMEM((ROWS,), jnp.int32),
        pltpu.VMEM((ROWS,), jnp.int32),
        pltpu.VMEM((ROWS,), jnp.int32),
        pltpu.VMEM((ROWS,), jnp.int32),
        pltpu.VMEM((ROWS, EMB), jnp.float32),
        pltpu.VMEM((ROWS, EMB), jnp.float32),
        pltpu.VMEM((ROWS, EMB), jnp.float32),
        pltpu.VMEM((ROWS, EMB), jnp.float32),
        pltpu.VMEM((E, EMB), jnp.float32),
        pltpu.VMEM((E, EMB), jnp.float32),
        pltpu.VMEM((E, 16), jnp.float32),
        pltpu.VMEM((E, 16), jnp.float32),
        pltpu.VMEM((E, NEG, 16), jnp.float32),
        pltpu.VMEM((E, NEG, 16), jnp.float32),
        pltpu.SemaphoreType.DMA,
        pltpu.SemaphoreType.DMA,
        pltpu.SemaphoreType.DMA,
        pltpu.SemaphoreType.DMA,
    ],
)

NEG_ROWS = B * NEG * 16 // 128
NEG_BLK = 4096
NSTEP = NEG_ROWS // NEG_BLK
POS_ROWS = B * 16 // 128


def _tail_body(posp_ref, w_ref, negp_ref, out_ref):
  i = pl.program_id(0)
  rg = lax.broadcasted_iota(jnp.int32, (128, 128), 0) // 16
  cj = lax.broadcasted_iota(jnp.int32, (128, 128), 1)
  gmat = jnp.where((rg == cj) & (cj < 8), jnp.float32(1.0), jnp.float32(0.0))

  @pl.when(i == 0)
  def _():
    pos_s = jnp.dot(posp_ref[...], gmat, preferred_element_type=jnp.float32)
    pos_l = jnp.log(jax.nn.sigmoid(pos_s) + 1e-10) * w_ref[...]
    out_ref[0, 0] = jnp.sum(pos_l)

  mask = (lax.broadcasted_iota(jnp.int32, (NEG_BLK, 128), 1) < 8)
  neg_s = jnp.dot(negp_ref[...], gmat, preferred_element_type=jnp.float32)
  neg_l = jnp.where(mask, jnp.log(jax.nn.sigmoid(-neg_s) + 1e-10),
                    jnp.float32(0.0))
  out_ref[0, 0] += jnp.sum(neg_l)

  @pl.when(i == NSTEP - 1)
  def _():
    out_ref[0, 0] = out_ref[0, 0] * jnp.float32(-1.0 / B)


_tail_call = pl.pallas_call(
    _tail_body,
    grid=(NSTEP,),
    in_specs=[
        pl.BlockSpec((POS_ROWS, 128), lambda i: (0, 0)),
        pl.BlockSpec((POS_ROWS, 128), lambda i: (0, 0)),
        pl.BlockSpec((NEG_BLK, 128), lambda i: (i, 0)),
    ],
    out_specs=pl.BlockSpec(memory_space=pltpu.SMEM),
    out_shape=jax.ShapeDtypeStruct((1, 1), jnp.float32),
)


def kernel(contexts, target, negatives, weights, W_in, W_out):
  ctx_t = contexts.astype(jnp.int32).T
  neg_t = negatives.astype(jnp.int32).T
  tgt = target.astype(jnp.int32)
  wint = W_in.T
  woutt = W_out.T
  tin = jnp.pad(W_in[VFULL:], ((0, HB - (VOCAB - VFULL)), (0, 0))).T
  tout = jnp.pad(W_out[VFULL:], ((0, HB - (VOCAB - VFULL)), (0, 0))).T
  win2 = _tr_call(wint, wint, tin).reshape(2 * OUTR, EMB)
  wout2 = _tr_call(woutt, woutt, tout).reshape(2 * OUTR, EMB)
  pos_part, neg_part = _sc_call(ctx_t, tgt, neg_t, win2, wout2)
  wpad = jnp.pad(weights.reshape(POS_ROWS, 8), ((0, 0), (0, 120)))
  loss = _tail_call(pos_part.reshape(POS_ROWS, 128), wpad,
                    neg_part.reshape(NEG_ROWS, 128))
  return loss.reshape(())


# speedup vs baseline: 8.0327x; 1.2948x over previous
import jax
import jax.numpy as jnp
from jax import lax
from jax.experimental import pallas as pl
from jax.experimental.pallas import tpu as pltpu
from jax.experimental.pallas import tpu_sc as plsc

VOCAB = 1000000
EMB = 64
B = 16384
CTX = 20
NEG = 20

NC = 2
NS = 16
NW = NC * NS
PER_W = B // NW
E = 16
NCHUNK = PER_W // E
ROWS = E * CTX
SUB = 4
RS = ROWS // SUB
K = EMB // 16

HB = 2048
LOG2HB = 11
NBLK = -(-VOCAB // (2 * HB))
OUTR = NBLK * HB
VFULL = (VOCAB // (2 * HB)) * 2 * HB
NFB = VOCAB // HB


def _tr_body(lo_ref, hi_ref, tail_ref, out_ref):
  sel = pl.program_id(0) == NBLK - 1
  lo = jnp.where(sel, tail_ref[...], lo_ref[...])
  out_ref[:, 0:64] = lo.T
  out_ref[:, 64:128] = hi_ref[...].T


_tr_call = pl.pallas_call(
    _tr_body,
    grid=(NBLK,),
    in_specs=[pl.BlockSpec((EMB, HB),
                           lambda i: (0, jnp.minimum(2 * i, NFB - 2))),
              pl.BlockSpec((EMB, HB),
                           lambda i: (0, jnp.minimum(2 * i + 1, NFB - 1))),
              pl.BlockSpec((EMB, HB), lambda i: (0, 0))],
    out_specs=pl.BlockSpec((HB, 128), lambda i: (i, 0)),
    out_shape=jax.ShapeDtypeStruct((OUTR, 128), jnp.float32),
)


def _xform(v):
  big = v >= jnp.int32(VFULL)
  small = (lax.shift_left(lax.shift_right_logical(v, LOG2HB + 1), LOG2HB + 1)
           + lax.shift_left(v & (HB - 1), 1)
           + (lax.shift_right_logical(v, LOG2HB) & 1))
  return jnp.where(big, v + v - jnp.int32(VFULL), small)


def _sc_body(ctxt_hbm, tgt_hbm, negt_hbm, win_hbm, wout_hbm,
             posp_hbm, negp_hbm,
             ctx_idx, neg_idx, tgt_idx, stage_c0, stage_c1, stage_n0, stage_n1,
             ctx_rows0, ctx_rows1, neg_rows0, neg_rows1,
             tgt_rows0, tgt_rows1,
             posp0, posp1, negp0, negp1,
             sem0, sem1, osem0, osem1):
  c = lax.axis_index("c")
  s = lax.axis_index("s")
  w = c * NS + s
  wbase = pl.multiple_of(w * PER_W, 8)

  pltpu.sync_copy(ctxt_hbm.at[:, pl.ds(wbase, PER_W)], ctx_idx)
  pltpu.sync_copy(negt_hbm.at[:, pl.ds(wbase, PER_W)], neg_idx)
  pltpu.sync_copy(tgt_hbm.at[pl.ds(wbase, PER_W)], tgt_idx)
  for i in range(PER_W // 16):
    tgt_idx[pl.ds(i * 16, 16)] = _xform(tgt_idx[pl.ds(i * 16, 16)])

  def stage(g, stage_c, stage_n):
    for cc in range(CTX):
      stage_c[pl.ds(cc * E, E)] = _xform(ctx_idx[cc, pl.ds(g * E, E)])
      stage_n[pl.ds(cc * E, E)] = _xform(neg_idx[cc, pl.ds(g * E, E)])

  def in_transfers(g, stage_c, stage_n, rows_ctx, rows_neg, rows_tgt, sem):
    ds = []
    for k in range(SUB):
      ds.append(pltpu.make_async_copy(
          win_hbm.at[stage_c.at[pl.ds(k * RS, RS)]],
          rows_ctx.at[pl.ds(k * RS, RS)], sem))
      ds.append(pltpu.make_async_copy(
          wout_hbm.at[stage_n.at[pl.ds(k * RS, RS)]],
          rows_neg.at[pl.ds(k * RS, RS)], sem))
    ds.append(pltpu.make_async_copy(
        wout_hbm.at[tgt_idx.at[pl.ds(pl.multiple_of(g * E, 8), E)]],
        rows_tgt, sem))
    return ds

  def out_transfers(g, posp, negp, osem):
    base = pl.multiple_of(w * PER_W + g * E, 8)
    return [
        pltpu.make_async_copy(posp, posp_hbm.at[pl.ds(base, E)], osem),
        pltpu.make_async_copy(negp, negp_hbm.at[pl.ds(base, E)], osem),
    ]

  def compute(g, rows_ctx, rows_neg, rows_tgt, posp, negp):
    def ebody(e, carry):
      acc = [rows_ctx[e, pl.ds(k * 16, 16)] for k in range(K)]
      for cc in range(1, CTX):
        for k in range(K):
          acc[k] = acc[k] + rows_ctx[cc * E + e, pl.ds(k * 16, 16)]
      emb = [a * jnp.float32(1.0 / CTX) for a in acc]
      p = emb[0] * rows_tgt[e, pl.ds(0, 16)]
      for k in range(1, K):
        p = p + emb[k] * rows_tgt[e, pl.ds(k * 16, 16)]
      posp[e, :] = p
      for j in range(NEG):
        q = emb[0] * rows_neg[j * E + e, pl.ds(0, 16)]
        for k in range(1, K):
          q = q + emb[k] * rows_neg[j * E + e, pl.ds(k * 16, 16)]
        negp[e, j, :] = q
      return carry

    lax.fori_loop(0, E, ebody, 0)

  stage(0, stage_c0, stage_n0)
  for d in in_transfers(0, stage_c0, stage_n0,
                        ctx_rows0, neg_rows0, tgt_rows0, sem0):
    d.start()
  stage(1, stage_c1, stage_n1)
  for d in in_transfers(1, stage_c1, stage_n1,
                        ctx_rows1, neg_rows1, tgt_rows1, sem1):
    d.start()

  def tbody(t, carry):
    for par, (stage_c, stage_n, rows_ctx, rows_neg, rows_tgt, posp, negp,
              sem, osem) in enumerate(
        ((stage_c0, stage_n0, ctx_rows0, neg_rows0, tgt_rows0, posp0, negp0,
          sem0, osem0),
         (stage_c1, stage_n1, ctx_rows1, neg_rows1, tgt_rows1, posp1, negp1,
          sem1, osem1))):
      g = t * 2 + par
      for d in in_transfers(g, stage_c, stage_n,
                            rows_ctx, rows_neg, rows_tgt, sem):
        d.wait()

      @pl.when(g >= 2)
      def _():
        for d in out_transfers(g - 2, posp, negp, osem):
          d.wait()

      compute(g, rows_ctx, rows_neg, rows_tgt, posp, negp)
      for d in out_transfers(g, posp, negp, osem):
        d.start()

      @pl.when(g + 2 < NCHUNK)
      def _():
        stage(g + 2, stage_c, stage_n)
        for d in in_transfers(g + 2, stage_c, stage_n,
                              rows_ctx, rows_neg, rows_tgt, sem):
          d.start()

    return carry

  lax.fori_loop(0, NCHUNK // 2, tbody, 0)

  for d in out_transfers(NCHUNK - 2, posp0, negp0, osem0):
    d.wait()
  for d in out_transfers(NCHUNK - 1, posp1, negp1, osem1):
    d.wait()


_sc_call = pl.kernel(
    _sc_body,
    out_type=(jax.ShapeDtypeStruct((B, 16), jnp.float32),
              jax.ShapeDtypeStruct((B, NEG, 16), jnp.float32)),
    mesh=plsc.VectorSubcoreMesh(core_axis_name="c", subcore_axis_name="s"),
    compiler_params=pltpu.CompilerParams(needs_layout_passes=False,
                                         use_tc_tiling_on_sc=False),
    scratch_types=[
        pltpu.VMEM((CTX, PER_W), jnp.int32),
        pltpu.VMEM((NEG, PER_W), jnp.int32),
        pltpu.VMEM((PER_W,), jnp.int32),
        pltpu.VMEM((ROWS,), jnp.int32),
        pltpu.VMEM((ROWS,), jnp.int32),
        pltpu.VMEM((ROWS,), jnp.int32),
        pltpu.VMEM((ROWS,), jnp.int32),
        pltpu.VMEM((ROWS, EMB), jnp.float32),
        pltpu.VMEM((ROWS, EMB), jnp.float32),
        pltpu.VMEM((ROWS, EMB), jnp.float32),
        pltpu.VMEM((ROWS, EMB), jnp.float32),
        pltpu.VMEM((E, EMB), jnp.float32),
        pltpu.VMEM((E, EMB), jnp.float32),
        pltpu.VMEM((E, 16), jnp.float32),
        pltpu.VMEM((E, 16), jnp.float32),
        pltpu.VMEM((E, NEG, 16), jnp.float32),
        pltpu.VMEM((E, NEG, 16), jnp.float32),
        pltpu.SemaphoreType.DMA,
        pltpu.SemaphoreType.DMA,
        pltpu.SemaphoreType.DMA,
        pltpu.SemaphoreType.DMA,
    ],
)

NEG_ROWS = B * NEG * 16 // 128
NEG_BLK = 4096
NSTEP = NEG_ROWS // NEG_BLK
POS_ROWS = B * 16 // 128


def _tail_body(posp_ref, w_ref, negp_ref, out_ref):
  i = pl.program_id(0)
  rg = lax.broadcasted_iota(jnp.int32, (128, 128), 0) // 16
  cj = lax.broadcasted_iota(jnp.int32, (128, 128), 1)
  gmat = jnp.where((rg == cj) & (cj < 8), jnp.float32(1.0), jnp.float32(0.0))

  @pl.when(i == 0)
  def _():
    pos_s = jnp.dot(posp_ref[...], gmat, preferred_element_type=jnp.float32)
    pos_l = jnp.log(jax.nn.sigmoid(pos_s) + 1e-10) * w_ref[...]
    out_ref[0, 0] = jnp.sum(pos_l)

  mask = (lax.broadcasted_iota(jnp.int32, (NEG_BLK, 128), 1) < 8)
  neg_s = jnp.dot(negp_ref[...], gmat, preferred_element_type=jnp.float32)
  neg_l = jnp.where(mask, jnp.log(jax.nn.sigmoid(-neg_s) + 1e-10),
                    jnp.float32(0.0))
  out_ref[0, 0] += jnp.sum(neg_l)

  @pl.when(i == NSTEP - 1)
  def _():
    out_ref[0, 0] = out_ref[0, 0] * jnp.float32(-1.0 / B)


_tail_call = pl.pallas_call(
    _tail_body,
    grid=(NSTEP,),
    in_specs=[
        pl.BlockSpec((POS_ROWS, 128), lambda i: (0, 0)),
        pl.BlockSpec((POS_ROWS, 128), lambda i: (0, 0)),
        pl.BlockSpec((NEG_BLK, 128), lambda i: (i, 0)),
    ],
    out_specs=pl.BlockSpec(memory_space=pltpu.SMEM),
    out_shape=jax.ShapeDtypeStruct((1, 1), jnp.float32),
)


def kernel(contexts, target, negatives, weights, W_in, W_out):
  ctx_t = contexts.astype(jnp.int32).T
  neg_t = negatives.astype(jnp.int32).T
  tgt = target.astype(jnp.int32)
  wint = W_in.T
  woutt = W_out.T
  tin = jnp.pad(W_in[VFULL:], ((0, HB - (VOCAB - VFULL)), (0, 0))).T
  tout = jnp.pad(W_out[VFULL:], ((0, HB - (VOCAB - VFULL)), (0, 0))).T
  win2 = _tr_call(wint, wint, tin).reshape(2 * OUTR, EMB)
  wout2 = _tr_call(woutt, woutt, tout).reshape(2 * OUTR, EMB)
  pos_part, neg_part = _sc_call(ctx_t, tgt, neg_t, win2, wout2)
  wpad = jnp.pad(weights.reshape(POS_ROWS, 8), ((0, 0), (0, 120)))
  loss = _tail_call(pos_part.reshape(POS_ROWS, 128), wpad,
                    neg_part.reshape(NEG_ROWS, 128))
  return loss.reshape(())


# speedup vs baseline: 9.6175x; 1.1973x over previous
import jax
import jax.numpy as jnp
from jax import lax
from jax.experimental import pallas as pl
from jax.experimental.pallas import tpu as pltpu
from jax.experimental.pallas import tpu_sc as plsc

VOCAB = 1000000
EMB = 64
B = 16384
CTX = 20
NEG = 20

NC = 2
NS = 16
NW = NC * NS
PER_W = B // NW
E = 16
NCHUNK = PER_W // E
ROWS = E * CTX
SUB = 4
RS = ROWS // SUB
K = EMB // 16

HB = 4096
LOG2HB = 12
NBLK = -(-VOCAB // (2 * HB))
OUTR = NBLK * HB
VFULL = (VOCAB // (2 * HB)) * 2 * HB
NFB = VOCAB // HB


def _tr_body(lo_ref, hi_ref, tail_ref, out_ref):
  sel = pl.program_id(0) == NBLK - 1
  lo = jnp.where(sel, tail_ref[...], lo_ref[...])
  out_ref[:, 0:64] = lo.T
  out_ref[:, 64:128] = hi_ref[...].T


_tr_call = pl.pallas_call(
    _tr_body,
    grid=(NBLK,),
    in_specs=[pl.BlockSpec((EMB, HB),
                           lambda i: (0, jnp.minimum(2 * i, NFB - 2))),
              pl.BlockSpec((EMB, HB),
                           lambda i: (0, jnp.minimum(2 * i + 1, NFB - 1))),
              pl.BlockSpec((EMB, HB), lambda i: (0, 0))],
    out_specs=pl.BlockSpec((HB, 128), lambda i: (i, 0)),
    out_shape=jax.ShapeDtypeStruct((OUTR, 128), jnp.float32),
)


def _xform(v):
  big = v >= jnp.int32(VFULL)
  small = (lax.shift_left(lax.shift_right_logical(v, LOG2HB + 1), LOG2HB + 1)
           + lax.shift_left(v & (HB - 1), 1)
           + (lax.shift_right_logical(v, LOG2HB) & 1))
  return jnp.where(big, v + v - jnp.int32(VFULL), small)


def _sc_body(ctxt_hbm, tgt_hbm, negt_hbm, win_hbm, wout_hbm,
             posp_hbm, negp_hbm,
             ctx_idx, neg_idx, tgt_idx, stage_c0, stage_c1, stage_n0, stage_n1,
             ctx_rows0, ctx_rows1, neg_rows0, neg_rows1,
             tgt_rows0, tgt_rows1,
             posp0, posp1, negp0, negp1,
             sem0, sem1, osem0, osem1):
  c = lax.axis_index("c")
  s = lax.axis_index("s")
  w = c * NS + s
  wbase = pl.multiple_of(w * PER_W, 8)

  pltpu.sync_copy(ctxt_hbm.at[:, pl.ds(wbase, PER_W)], ctx_idx)
  pltpu.sync_copy(negt_hbm.at[:, pl.ds(wbase, PER_W)], neg_idx)
  pltpu.sync_copy(tgt_hbm.at[pl.ds(wbase, PER_W)], tgt_idx)
  for i in range(PER_W // 16):
    tgt_idx[pl.ds(i * 16, 16)] = _xform(tgt_idx[pl.ds(i * 16, 16)])

  def stage(g, stage_c, stage_n):
    for cc in range(CTX):
      stage_c[pl.ds(cc * E, E)] = _xform(ctx_idx[cc, pl.ds(g * E, E)])
      stage_n[pl.ds(cc * E, E)] = _xform(neg_idx[cc, pl.ds(g * E, E)])

  def in_transfers(g, stage_c, stage_n, rows_ctx, rows_neg, rows_tgt, sem):
    ds = []
    for k in range(SUB):
      ds.append(pltpu.make_async_copy(
          win_hbm.at[stage_c.at[pl.ds(k * RS, RS)]],
          rows_ctx.at[pl.ds(k * RS, RS)], sem))
      ds.append(pltpu.make_async_copy(
          wout_hbm.at[stage_n.at[pl.ds(k * RS, RS)]],
          rows_neg.at[pl.ds(k * RS, RS)], sem))
    ds.append(pltpu.make_async_copy(
        wout_hbm.at[tgt_idx.at[pl.ds(pl.multiple_of(g * E, 8), E)]],
        rows_tgt, sem))
    return ds

  def out_transfers(g, posp, negp, osem):
    base = pl.multiple_of(w * PER_W + g * E, 8)
    return [
        pltpu.make_async_copy(posp, posp_hbm.at[pl.ds(base, E)], osem),
        pltpu.make_async_copy(negp, negp_hbm.at[pl.ds(base, E)], osem),
    ]

  def compute(g, rows_ctx, rows_neg, rows_tgt, posp, negp):
    def ebody(e, carry):
      acc = [rows_ctx[e, pl.ds(k * 16, 16)] for k in range(K)]
      for cc in range(1, CTX):
        for k in range(K):
          acc[k] = acc[k] + rows_ctx[cc * E + e, pl.ds(k * 16, 16)]
      emb = [a * jnp.float32(1.0 / CTX) for a in acc]
      p = emb[0] * rows_tgt[e, pl.ds(0, 16)]
      for k in range(1, K):
        p = p + emb[k] * rows_tgt[e, pl.ds(k * 16, 16)]
      posp[e, :] = p
      for j in range(NEG):
        q = emb[0] * rows_neg[j * E + e, pl.ds(0, 16)]
        for k in range(1, K):
          q = q + emb[k] * rows_neg[j * E + e, pl.ds(k * 16, 16)]
        negp[e, j, :] = q
      return carry

    lax.fori_loop(0, E, ebody, 0)

  stage(0, stage_c0, stage_n0)
  for d in in_transfers(0, stage_c0, stage_n0,
                        ctx_rows0, neg_rows0, tgt_rows0, sem0):
    d.start()
  stage(1, stage_c1, stage_n1)
  for d in in_transfers(1, stage_c1, stage_n1,
                        ctx_rows1, neg_rows1, tgt_rows1, sem1):
    d.start()

  def tbody(t, carry):
    for par, (stage_c, stage_n, rows_ctx, rows_neg, rows_tgt, posp, negp,
              sem, osem) in enumerate(
        ((stage_c0, stage_n0, ctx_rows0, neg_rows0, tgt_rows0, posp0, negp0,
          sem0, osem0),
         (stage_c1, stage_n1, ctx_rows1, neg_rows1, tgt_rows1, posp1, negp1,
          sem1, osem1))):
      g = t * 2 + par
      for d in in_transfers(g, stage_c, stage_n,
                            rows_ctx, rows_neg, rows_tgt, sem):
        d.wait()

      @pl.when(g >= 2)
      def _():
        for d in out_transfers(g - 2, posp, negp, osem):
          d.wait()

      compute(g, rows_ctx, rows_neg, rows_tgt, posp, negp)
      for d in out_transfers(g, posp, negp, osem):
        d.start()

      @pl.when(g + 2 < NCHUNK)
      def _():
        stage(g + 2, stage_c, stage_n)
        for d in in_transfers(g + 2, stage_c, stage_n,
                              rows_ctx, rows_neg, rows_tgt, sem):
          d.start()

    return carry

  lax.fori_loop(0, NCHUNK // 2, tbody, 0)

  for d in out_transfers(NCHUNK - 2, posp0, negp0, osem0):
    d.wait()
  for d in out_transfers(NCHUNK - 1, posp1, negp1, osem1):
    d.wait()


_sc_call = pl.kernel(
    _sc_body,
    out_type=(jax.ShapeDtypeStruct((B, 16), jnp.float32),
              jax.ShapeDtypeStruct((B, NEG, 16), jnp.float32)),
    mesh=plsc.VectorSubcoreMesh(core_axis_name="c", subcore_axis_name="s"),
    compiler_params=pltpu.CompilerParams(needs_layout_passes=False,
                                         use_tc_tiling_on_sc=False),
    scratch_types=[
        pltpu.VMEM((CTX, PER_W), jnp.int32),
        pltpu.VMEM((NEG, PER_W), jnp.int32),
        pltpu.VMEM((PER_W,), jnp.int32),
        pltpu.VMEM((ROWS,), jnp.int32),
        pltpu.VMEM((ROWS,), jnp.int32),
        pltpu.VMEM((ROWS,), jnp.int32),
        pltpu.VMEM((ROWS,), jnp.int32),
        pltpu.VMEM((ROWS, EMB), jnp.float32),
        pltpu.VMEM((ROWS, EMB), jnp.float32),
        pltpu.VMEM((ROWS, EMB), jnp.float32),
        pltpu.VMEM((ROWS, EMB), jnp.float32),
        pltpu.VMEM((E, EMB), jnp.float32),
        pltpu.VMEM((E, EMB), jnp.float32),
        pltpu.VMEM((E, 16), jnp.float32),
        pltpu.VMEM((E, 16), jnp.float32),
        pltpu.VMEM((E, NEG, 16), jnp.float32),
        pltpu.VMEM((E, NEG, 16), jnp.float32),
        pltpu.SemaphoreType.DMA,
        pltpu.SemaphoreType.DMA,
        pltpu.SemaphoreType.DMA,
        pltpu.SemaphoreType.DMA,
    ],
)

NEG_ROWS = B * NEG * 16 // 128
NEG_BLK = 4096
NSTEP = NEG_ROWS // NEG_BLK
POS_ROWS = B * 16 // 128


def _tail_body(posp_ref, w_ref, negp_ref, out_ref):
  i = pl.program_id(0)
  rg = lax.broadcasted_iota(jnp.int32, (128, 128), 0) // 16
  cj = lax.broadcasted_iota(jnp.int32, (128, 128), 1)
  gmat = jnp.where((rg == cj) & (cj < 8), jnp.float32(1.0), jnp.float32(0.0))

  @pl.when(i == 0)
  def _():
    pos_s = jnp.dot(posp_ref[...], gmat, preferred_element_type=jnp.float32)
    pos_l = jnp.log(jax.nn.sigmoid(pos_s) + 1e-10) * w_ref[...]
    out_ref[0, 0] = jnp.sum(pos_l)

  mask = (lax.broadcasted_iota(jnp.int32, (NEG_BLK, 128), 1) < 8)
  neg_s = jnp.dot(negp_ref[...], gmat, preferred_element_type=jnp.float32)
  neg_l = jnp.where(mask, jnp.log(jax.nn.sigmoid(-neg_s) + 1e-10),
                    jnp.float32(0.0))
  out_ref[0, 0] += jnp.sum(neg_l)

  @pl.when(i == NSTEP - 1)
  def _():
    out_ref[0, 0] = out_ref[0, 0] * jnp.float32(-1.0 / B)


_tail_call = pl.pallas_call(
    _tail_body,
    grid=(NSTEP,),
    in_specs=[
        pl.BlockSpec((POS_ROWS, 128), lambda i: (0, 0)),
        pl.BlockSpec((POS_ROWS, 128), lambda i: (0, 0)),
        pl.BlockSpec((NEG_BLK, 128), lambda i: (i, 0)),
    ],
    out_specs=pl.BlockSpec(memory_space=pltpu.SMEM),
    out_shape=jax.ShapeDtypeStruct((1, 1), jnp.float32),
)


def kernel(contexts, target, negatives, weights, W_in, W_out):
  ctx_t = contexts.astype(jnp.int32).T
  neg_t = negatives.astype(jnp.int32).T
  tgt = target.astype(jnp.int32)
  wint = W_in.T
  woutt = W_out.T
  tin = jnp.pad(W_in[VFULL:], ((0, HB - (VOCAB - VFULL)), (0, 0))).T
  tout = jnp.pad(W_out[VFULL:], ((0, HB - (VOCAB - VFULL)), (0, 0))).T
  win2 = _tr_call(wint, wint, tin).reshape(2 * OUTR, EMB)
  wout2 = _tr_call(woutt, woutt, tout).reshape(2 * OUTR, EMB)
  pos_part, neg_part = _sc_call(ctx_t, tgt, neg_t, win2, wout2)
  wpad = jnp.pad(weights.reshape(POS_ROWS, 8), ((0, 0), (0, 120)))
  loss = _tail_call(pos_part.reshape(POS_ROWS, 128), wpad,
                    neg_part.reshape(NEG_ROWS, 128))
  return loss.reshape(())


# speedup vs baseline: 10.6317x; 1.1055x over previous
import jax
import jax.numpy as jnp
from jax import lax
from jax.experimental import pallas as pl
from jax.experimental.pallas import tpu as pltpu
from jax.experimental.pallas import tpu_sc as plsc

VOCAB = 1000000
EMB = 64
B = 16384
CTX = 20
NEG = 20

NC = 2
NS = 16
NW = NC * NS
PER_W = B // NW
E = 16
NCHUNK = PER_W // E
ROWS = E * CTX
SUB = 4
RS = ROWS // SUB
K = EMB // 16

HB = 8192
LOG2HB = 13
NBLK = -(-VOCAB // (2 * HB))
OUTR = NBLK * HB
VFULL = (VOCAB // (2 * HB)) * 2 * HB
NFB = VOCAB // HB


def _tr_body(lo_ref, hi_ref, tail_ref, out_ref):
  sel = pl.program_id(0) == NBLK - 1
  lo = jnp.where(sel, tail_ref[...], lo_ref[...])
  out_ref[:, 0:64] = lo.T
  out_ref[:, 64:128] = hi_ref[...].T


_tr_call = pl.pallas_call(
    _tr_body,
    grid=(NBLK,),
    in_specs=[pl.BlockSpec((EMB, HB),
                           lambda i: (0, jnp.minimum(2 * i, NFB - 2))),
              pl.BlockSpec((EMB, HB),
                           lambda i: (0, jnp.minimum(2 * i + 1, NFB - 1))),
              pl.BlockSpec((EMB, HB), lambda i: (0, 0))],
    out_specs=pl.BlockSpec((HB, 128), lambda i: (i, 0)),
    out_shape=jax.ShapeDtypeStruct((OUTR, 128), jnp.float32),
)


def _xform(v):
  big = v >= jnp.int32(VFULL)
  small = (lax.shift_left(lax.shift_right_logical(v, LOG2HB + 1), LOG2HB + 1)
           + lax.shift_left(v & (HB - 1), 1)
           + (lax.shift_right_logical(v, LOG2HB) & 1))
  return jnp.where(big, v + v - jnp.int32(VFULL), small)


def _sc_body(ctxt_hbm, tgt_hbm, negt_hbm, win_hbm, wout_hbm,
             posp_hbm, negp_hbm,
             ctx_idx, neg_idx, tgt_idx, stage_c0, stage_c1, stage_n0, stage_n1,
             ctx_rows0, ctx_rows1, neg_rows0, neg_rows1,
             tgt_rows0, tgt_rows1,
             posp0, posp1, negp0, negp1,
             sem0, sem1, osem0, osem1):
  c = lax.axis_index("c")
  s = lax.axis_index("s")
  w = c * NS + s
  wbase = pl.multiple_of(w * PER_W, 8)

  pltpu.sync_copy(ctxt_hbm.at[:, pl.ds(wbase, PER_W)], ctx_idx)
  pltpu.sync_copy(negt_hbm.at[:, pl.ds(wbase, PER_W)], neg_idx)
  pltpu.sync_copy(tgt_hbm.at[pl.ds(wbase, PER_W)], tgt_idx)
  for i in range(PER_W // 16):
    tgt_idx[pl.ds(i * 16, 16)] = _xform(tgt_idx[pl.ds(i * 16, 16)])

  def stage(g, stage_c, stage_n):
    for cc in range(CTX):
      stage_c[pl.ds(cc * E, E)] = _xform(ctx_idx[cc, pl.ds(g * E, E)])
      stage_n[pl.ds(cc * E, E)] = _xform(neg_idx[cc, pl.ds(g * E, E)])

  def in_transfers(g, stage_c, stage_n, rows_ctx, rows_neg, rows_tgt, sem):
    ds = []
    for k in range(SUB):
      ds.append(pltpu.make_async_copy(
          win_hbm.at[stage_c.at[pl.ds(k * RS, RS)]],
          rows_ctx.at[pl.ds(k * RS, RS)], sem))
      ds.append(pltpu.make_async_copy(
          wout_hbm.at[stage_n.at[pl.ds(k * RS, RS)]],
          rows_neg.at[pl.ds(k * RS, RS)], sem))
    ds.append(pltpu.make_async_copy(
        wout_hbm.at[tgt_idx.at[pl.ds(pl.multiple_of(g * E, 8), E)]],
        rows_tgt, sem))
    return ds

  def out_transfers(g, posp, negp, osem):
    base = pl.multiple_of(w * PER_W + g * E, 8)
    return [
        pltpu.make_async_copy(posp, posp_hbm.at[pl.ds(base, E)], osem),
        pltpu.make_async_copy(negp, negp_hbm.at[pl.ds(base, E)], osem),
    ]

  def compute(g, rows_ctx, rows_neg, rows_tgt, posp, negp):
    def ebody(e, carry):
      acc = [rows_ctx[e, pl.ds(k * 16, 16)] for k in range(K)]
      for cc in range(1, CTX):
        for k in range(K):
          acc[k] = acc[k] + rows_ctx[cc * E + e, pl.ds(k * 16, 16)]
      emb = [a * jnp.float32(1.0 / CTX) for a in acc]
      p = emb[0] * rows_tgt[e, pl.ds(0, 16)]
      for k in range(1, K):
        p = p + emb[k] * rows_tgt[e, pl.ds(k * 16, 16)]
      posp[e, :] = p
      for j in range(NEG):
        q = emb[0] * rows_neg[j * E + e, pl.ds(0, 16)]
        for k in range(1, K):
          q = q + emb[k] * rows_neg[j * E + e, pl.ds(k * 16, 16)]
        negp[e, j, :] = q
      return carry

    lax.fori_loop(0, E, ebody, 0)

  stage(0, stage_c0, stage_n0)
  for d in in_transfers(0, stage_c0, stage_n0,
                        ctx_rows0, neg_rows0, tgt_rows0, sem0):
    d.start()
  stage(1, stage_c1, stage_n1)
  for d in in_transfers(1, stage_c1, stage_n1,
                        ctx_rows1, neg_rows1, tgt_rows1, sem1):
    d.start()

  def tbody(t, carry):
    for par, (stage_c, stage_n, rows_ctx, rows_neg, rows_tgt, posp, negp,
              sem, osem) in enumerate(
        ((stage_c0, stage_n0, ctx_rows0, neg_rows0, tgt_rows0, posp0, negp0,
          sem0, osem0),
         (stage_c1, stage_n1, ctx_rows1, neg_rows1, tgt_rows1, posp1, negp1,
          sem1, osem1))):
      g = t * 2 + par
      for d in in_transfers(g, stage_c, stage_n,
                            rows_ctx, rows_neg, rows_tgt, sem):
        d.wait()

      @pl.when(g >= 2)
      def _():
        for d in out_transfers(g - 2, posp, negp, osem):
          d.wait()

      compute(g, rows_ctx, rows_neg, rows_tgt, posp, negp)
      for d in out_transfers(g, posp, negp, osem):
        d.start()

      @pl.when(g + 2 < NCHUNK)
      def _():
        stage(g + 2, stage_c, stage_n)
        for d in in_transfers(g + 2, stage_c, stage_n,
                              rows_ctx, rows_neg, rows_tgt, sem):
          d.start()

    return carry

  lax.fori_loop(0, NCHUNK // 2, tbody, 0)

  for d in out_transfers(NCHUNK - 2, posp0, negp0, osem0):
    d.wait()
  for d in out_transfers(NCHUNK - 1, posp1, negp1, osem1):
    d.wait()


_sc_call = pl.kernel(
    _sc_body,
    out_type=(jax.ShapeDtypeStruct((B, 16), jnp.float32),
              jax.ShapeDtypeStruct((B, NEG, 16), jnp.float32)),
    mesh=plsc.VectorSubcoreMesh(core_axis_name="c", subcore_axis_name="s"),
    compiler_params=pltpu.CompilerParams(needs_layout_passes=False,
                                         use_tc_tiling_on_sc=False),
    scratch_types=[
        pltpu.VMEM((CTX, PER_W), jnp.int32),
        pltpu.VMEM((NEG, PER_W), jnp.int32),
        pltpu.VMEM((PER_W,), jnp.int32),
        pltpu.VMEM((ROWS,), jnp.int32),
        pltpu.VMEM((ROWS,), jnp.int32),
        pltpu.VMEM((ROWS,), jnp.int32),
        pltpu.VMEM((ROWS,), jnp.int32),
        pltpu.VMEM((ROWS, EMB), jnp.float32),
        pltpu.VMEM((ROWS, EMB), jnp.float32),
        pltpu.VMEM((ROWS, EMB), jnp.float32),
        pltpu.VMEM((ROWS, EMB), jnp.float32),
        pltpu.VMEM((E, EMB), jnp.float32),
        pltpu.VMEM((E, EMB), jnp.float32),
        pltpu.VMEM((E, 16), jnp.float32),
        pltpu.VMEM((E, 16), jnp.float32),
        pltpu.VMEM((E, NEG, 16), jnp.float32),
        pltpu.VMEM((E, NEG, 16), jnp.float32),
        pltpu.SemaphoreType.DMA,
        pltpu.SemaphoreType.DMA,
        pltpu.SemaphoreType.DMA,
        pltpu.SemaphoreType.DMA,
    ],
)

NEG_ROWS = B * NEG * 16 // 128
NEG_BLK = 4096
NSTEP = NEG_ROWS // NEG_BLK
POS_ROWS = B * 16 // 128


def _tail_body(posp_ref, w_ref, negp_ref, out_ref):
  i = pl.program_id(0)
  rg = lax.broadcasted_iota(jnp.int32, (128, 128), 0) // 16
  cj = lax.broadcasted_iota(jnp.int32, (128, 128), 1)
  gmat = jnp.where((rg == cj) & (cj < 8), jnp.float32(1.0), jnp.float32(0.0))

  @pl.when(i == 0)
  def _():
    pos_s = jnp.dot(posp_ref[...], gmat, preferred_element_type=jnp.float32)
    pos_l = jnp.log(jax.nn.sigmoid(pos_s) + 1e-10) * w_ref[...]
    out_ref[0, 0] = jnp.sum(pos_l)

  mask = (lax.broadcasted_iota(jnp.int32, (NEG_BLK, 128), 1) < 8)
  neg_s = jnp.dot(negp_ref[...], gmat, preferred_element_type=jnp.float32)
  neg_l = jnp.where(mask, jnp.log(jax.nn.sigmoid(-neg_s) + 1e-10),
                    jnp.float32(0.0))
  out_ref[0, 0] += jnp.sum(neg_l)

  @pl.when(i == NSTEP - 1)
  def _():
    out_ref[0, 0] = out_ref[0, 0] * jnp.float32(-1.0 / B)


_tail_call = pl.pallas_call(
    _tail_body,
    grid=(NSTEP,),
    in_specs=[
        pl.BlockSpec((POS_ROWS, 128), lambda i: (0, 0)),
        pl.BlockSpec((POS_ROWS, 128), lambda i: (0, 0)),
        pl.BlockSpec((NEG_BLK, 128), lambda i: (i, 0)),
    ],
    out_specs=pl.BlockSpec(memory_space=pltpu.SMEM),
    out_shape=jax.ShapeDtypeStruct((1, 1), jnp.float32),
)


def kernel(contexts, target, negatives, weights, W_in, W_out):
  ctx_t = contexts.astype(jnp.int32).T
  neg_t = negatives.astype(jnp.int32).T
  tgt = target.astype(jnp.int32)
  wint = W_in.T
  woutt = W_out.T
  tin = jnp.pad(W_in[VFULL:], ((0, HB - (VOCAB - VFULL)), (0, 0))).T
  tout = jnp.pad(W_out[VFULL:], ((0, HB - (VOCAB - VFULL)), (0, 0))).T
  win2 = _tr_call(wint, wint, tin).reshape(2 * OUTR, EMB)
  wout2 = _tr_call(woutt, woutt, tout).reshape(2 * OUTR, EMB)
  pos_part, neg_part = _sc_call(ctx_t, tgt, neg_t, win2, wout2)
  wpad = jnp.pad(weights.reshape(POS_ROWS, 8), ((0, 0), (0, 120)))
  loss = _tail_call(pos_part.reshape(POS_ROWS, 128), wpad,
                    neg_part.reshape(NEG_ROWS, 128))
  return loss.reshape(())
